# baseline (speedup 1.0000x reference)
import functools

import jax
import jax.numpy as jnp
from jax import lax
from jax.experimental import pallas as pl
from jax.experimental.pallas import tpu as pltpu
from jax.experimental.pallas import tpu_sc as plsc

D_MODEL = 64
_NC = 2
_NS = 16
_NW = _NC * _NS
_GATHER = 128
_K = 4
_CHUNK = _K * _GATHER


@functools.cache
def _make_embed(B, V):
    rows_per_w = B // _NW
    n_chunks = rows_per_w // _CHUNK
    idx_rows_per_w = rows_per_w // _GATHER
    mesh = plsc.VectorSubcoreMesh(core_axis_name="c", subcore_axis_name="s")

    @functools.partial(
        pl.kernel,
        out_type=jax.ShapeDtypeStruct((B, D_MODEL), jnp.float32),
        mesh=mesh,
        scratch_types=[
            pltpu.VMEM((_K, _GATHER), jnp.int32),
            pltpu.VMEM((_CHUNK, D_MODEL), jnp.float32),
            pltpu.SemaphoreType.DMA,
        ],
        compiler_params=pltpu.CompilerParams(use_tc_tiling_on_sc=False),
    )
    def embed(idx_hbm, table_hbm, out_hbm, idx_v, rows_v, sem):
        wid = lax.axis_index("s") * _NC + lax.axis_index("c")
        row0 = wid * idx_rows_per_w
        base = wid * rows_per_w

        @pl.loop(0, n_chunks)
        def _chunk(c):
            pltpu.sync_copy(idx_hbm.at[pl.ds(row0 + c * _K, _K)], idx_v)
            copies = [
                pltpu.async_copy(
                    table_hbm.at[idx_v.at[j]],
                    rows_v.at[pl.ds(j * _GATHER, _GATHER)],
                    sem,
                )
                for j in range(_K)
            ]
            for cp in copies:
                cp.wait()
            pltpu.sync_copy(rows_v, out_hbm.at[pl.ds(base + c * _CHUNK, _CHUNK)])

    return embed


def kernel(idx, weight):
    B = idx.size
    idx_flat = idx.reshape(B // _GATHER, _GATHER)
    out = _make_embed(B, weight.shape[0])(idx_flat, weight)
    return out.reshape(idx.shape + (weight.shape[1],))
